# Initial kernel scaffold; baseline (speedup 1.0000x reference)
import jax
import jax.numpy as jnp
from jax.experimental import pallas as pl


def kernel(x, indices, table):
    raise NotImplementedError("write your pallas kernel here")


# speedup vs baseline: 1.8674x; 1.8674x over previous
import jax
import jax.numpy as jnp
from jax.experimental import pallas as pl

_R = 256


def _tc_body(idx_ref, x_ref, table_ref, o_ref):
    idx = idx_ref[0, 0, :]
    iota = jax.lax.broadcasted_iota(jnp.int32, (_R, 64), 1)
    onehot = (idx[:, None] == iota).astype(jnp.float32)
    emb = jnp.dot(onehot, table_ref[...], preferred_element_type=jnp.float32)
    o_ref[...] = x_ref[...] + emb


def kernel(x, indices, table):
    B, S, D = x.shape
    V = table.shape[0]
    N = B * S
    assert N % _R == 0
    x2 = x.reshape(N, D)
    idx2 = indices.reshape(N // _R, 1, _R).astype(jnp.int32)
    out = pl.pallas_call(
        _tc_body,
        grid=(N // _R,),
        in_specs=[
            pl.BlockSpec((1, 1, _R), lambda i: (i, 0, 0)),
            pl.BlockSpec((_R, D), lambda i: (i, 0)),
            pl.BlockSpec((V, D), lambda i: (0, 0)),
        ],
        out_specs=pl.BlockSpec((_R, D), lambda i: (i, 0)),
        out_shape=jax.ShapeDtypeStruct((N, D), jnp.float32),
    )(idx2, x2, table)
    return out.reshape(B, S, D)
